# Initial kernel scaffold; baseline (speedup 1.0000x reference)
import functools

import jax
import jax.numpy as jnp
from jax import lax
from jax.experimental import pallas as pl
from jax.experimental.pallas import tpu as pltpu
from jax.experimental.pallas import tpu_sc as plsc

N = 10000
E = 320000
D = 128
STRIDE = 4
M = N // STRIDE
DOWN = 128

NC = 2
NS = 16
NW = NC * NS
EPW = E // NW
K = 80
NB = EPW // K

MPAD = 2560
SM = MPAD // NS
NROWS = N + M
NPAD = 12544


def _matmul_body(x_ref, w_ref, o_ref):
    o_ref[...] = jnp.dot(x_ref[...], w_ref[...],
                         preferred_element_type=jnp.float32)


def _combine_body(p_ref, o_ref):
    o_ref[...] = p_ref[0] + p_ref[1]


def _sc_body(qall, ep3, en3, zrows, out, ep_v, en_v, en2_v, q_v, b2_v,
             acc, sem1, sem2):
    cid = lax.axis_index("c")
    sid = lax.axis_index("s")
    wid = sid * NC + cid

    pltpu.sync_copy(zrows, acc.at[pl.ds(sid * SM, SM)])
    plsc.subcore_barrier()

    def blk(j, carry):
        pltpu.sync_copy(ep3.at[wid, j], ep_v)
        pltpu.sync_copy(en3.at[wid, j], en_v)
        for c in range(K // 16):
            en2_v[pl.ds(c * 16, 16)] = en_v[pl.ds(c * 16, 16)] + N
        cp1 = pltpu.async_copy(qall.at[ep_v], q_v, sem1)
        cp2 = pltpu.async_copy(qall.at[en2_v], b2_v, sem2)
        cp1.wait()
        cp2.wait()

        def row(r, carry2):
            for c in range(DOWN // 16):
                s = pl.ds(c * 16, 16)
                q_v[r, s] = jnp.maximum(q_v[r, s] - b2_v[r, s], 0.0)
            return carry2

        lax.fori_loop(0, K, row, 0)
        pltpu.sync_copy(q_v, acc.at[en_v], add=True)
        return carry

    lax.fori_loop(0, NB, blk, 0)
    plsc.subcore_barrier()
    pltpu.sync_copy(acc.at[pl.ds(sid * SM, SM)],
                    out.at[cid, pl.ds(sid * SM, SM)])


_sc_kernel = functools.partial(
    pl.kernel,
    out_type=jax.ShapeDtypeStruct((NC, MPAD, DOWN), jnp.float32),
    mesh=plsc.VectorSubcoreMesh(core_axis_name="c", subcore_axis_name="s"),
    scratch_types=[
        pltpu.VMEM((K,), jnp.int32),
        pltpu.VMEM((K,), jnp.int32),
        pltpu.VMEM((K,), jnp.int32),
        pltpu.VMEM((K, DOWN), jnp.float32),
        pltpu.VMEM((K, DOWN), jnp.float32),
        pltpu.VMEM_SHARED((MPAD, DOWN), jnp.float32),
        pltpu.SemaphoreType.DMA,
        pltpu.SemaphoreType.DMA,
    ],
)(_sc_body)


@jax.jit
def kernel(point_bxyz, point_feat, e_point, e_new, W, b):
    new_bxyz = point_bxyz[::STRIDE]
    xyz = point_bxyz[:, 1:4]
    nxyz = new_bxyz[:, 1:4]

    x_top = jnp.concatenate(
        [point_feat, xyz, jnp.ones((N, 1), jnp.float32)], axis=1)
    x_new = jnp.concatenate(
        [jnp.zeros((M, D), jnp.float32), nxyz,
         jnp.zeros((M, 1), jnp.float32)], axis=1)
    x_pad = jnp.zeros((NPAD - NROWS, D + 4), jnp.float32)
    x_all = jnp.concatenate([x_top, x_new, x_pad], axis=0)
    w_cat = jnp.concatenate(
        [W, b[None, :], jnp.zeros((1, DOWN), jnp.float32)], axis=0)

    qall = pl.pallas_call(
        _matmul_body,
        out_shape=jax.ShapeDtypeStruct((NPAD, DOWN), jnp.float32),
    )(x_all, w_cat)

    ep3 = e_point.reshape(NW, NB, K)
    en3 = e_new.reshape(NW, NB, K)
    zrows = jnp.zeros((SM, DOWN), jnp.float32)

    partials = _sc_kernel(qall, ep3, en3, zrows)

    feat_pad = pl.pallas_call(
        _combine_body,
        out_shape=jax.ShapeDtypeStruct((MPAD, DOWN), jnp.float32),
    )(partials)
    new_feat = feat_pad[:M]

    return (new_bxyz, new_feat, e_point, e_new)


# speedup vs baseline: 49.4922x; 49.4922x over previous
import functools

import jax
import jax.numpy as jnp
from jax import lax
from jax.experimental import pallas as pl
from jax.experimental.pallas import tpu as pltpu
from jax.experimental.pallas import tpu_sc as plsc

N = 10000
E = 320000
D = 128
STRIDE = 4
M = N // STRIDE
DOWN = 128

NC = 2
NS = 16
NW = NC * NS
EPW = E // NW
K = 80
NB = EPW // K

MPAD = 2560
SM = MPAD // NS
NROWS = N + M
NPAD = 12544


def _matmul_body(x_ref, w_ref, o_ref):
    o_ref[...] = jnp.dot(x_ref[...], w_ref[...],
                         preferred_element_type=jnp.float32)


def _combine_body(p_ref, o_ref):
    o_ref[...] = p_ref[0] + p_ref[1]


def _sc_body(qall, ep3, en3, zrows, out, ep_v, en_v, en2_v, q_v, b2_v,
             acc, sem1, sem2):
    cid = lax.axis_index("c")
    sid = lax.axis_index("s")
    wid = sid * NC + cid

    pltpu.sync_copy(zrows, acc.at[pl.ds(sid * SM, SM)])
    plsc.subcore_barrier()

    def blk(j, carry):
        pltpu.sync_copy(ep3.at[wid, j], ep_v)
        pltpu.sync_copy(en3.at[wid, j], en_v)
        for c in range(K // 16):
            en2_v[pl.ds(c * 16, 16)] = en_v[pl.ds(c * 16, 16)] + N
        cp1 = pltpu.async_copy(qall.at[ep_v], q_v, sem1)
        cp2 = pltpu.async_copy(qall.at[en2_v], b2_v, sem2)
        cp1.wait()
        cp2.wait()

        def row(r, carry2):
            for c in range(DOWN // 16):
                s = pl.ds(c * 16, 16)
                q_v[r, s] = jnp.maximum(q_v[r, s] - b2_v[r, s], 0.0)
            return carry2

        lax.fori_loop(0, K, row, 0)
        pltpu.sync_copy(q_v, acc.at[en_v], add=True)
        return carry

    lax.fori_loop(0, NB, blk, 0)
    plsc.subcore_barrier()
    pltpu.sync_copy(acc.at[pl.ds(sid * SM, SM)],
                    out.at[cid, pl.ds(sid * SM, SM)])


_sc_kernel = functools.partial(
    pl.kernel,
    out_type=jax.ShapeDtypeStruct((NC, MPAD, DOWN), jnp.float32),
    mesh=plsc.VectorSubcoreMesh(core_axis_name="c", subcore_axis_name="s"),
    scratch_types=[
        pltpu.VMEM((K,), jnp.int32),
        pltpu.VMEM((K,), jnp.int32),
        pltpu.VMEM((K,), jnp.int32),
        pltpu.VMEM((K, DOWN), jnp.float32),
        pltpu.VMEM((K, DOWN), jnp.float32),
        pltpu.VMEM_SHARED((MPAD, DOWN), jnp.float32),
        pltpu.SemaphoreType.DMA,
        pltpu.SemaphoreType.DMA,
    ],
)(_sc_body)


@jax.jit
def kernel(point_bxyz, point_feat, e_point, e_new, W, b):
    new_bxyz = point_bxyz[::STRIDE]
    xyz = point_bxyz[:, 1:4]
    nxyz = new_bxyz[:, 1:4]

    x_top = jnp.concatenate(
        [point_feat, xyz, jnp.ones((N, 1), jnp.float32)], axis=1)
    x_new = jnp.concatenate(
        [jnp.zeros((M, D), jnp.float32), nxyz,
         jnp.zeros((M, 1), jnp.float32)], axis=1)
    x_pad = jnp.zeros((NPAD - NROWS, D + 4), jnp.float32)
    x_all = jnp.concatenate([x_top, x_new, x_pad], axis=0)
    w_cat = jnp.concatenate([W, b[None, :]], axis=0)

    qall = pl.pallas_call(
        _matmul_body,
        out_shape=jax.ShapeDtypeStruct((NPAD, DOWN), jnp.float32),
    )(x_all, w_cat)

    ep3 = e_point.reshape(NW, NB, K)
    en3 = e_new.reshape(NW, NB, K)
    zrows = jnp.zeros((SM, DOWN), jnp.float32)

    partials = _sc_kernel(qall, ep3, en3, zrows)

    feat_pad = pl.pallas_call(
        _combine_body,
        out_shape=jax.ShapeDtypeStruct((MPAD, DOWN), jnp.float32),
    )(partials)
    new_feat = feat_pad[:M]

    return (new_bxyz, new_feat, e_point, e_new)


# speedup vs baseline: 107.1630x; 2.1652x over previous
import functools

import jax
import jax.numpy as jnp
from jax import lax
from jax.experimental import pallas as pl
from jax.experimental.pallas import tpu as pltpu
from jax.experimental.pallas import tpu_sc as plsc

N = 10000
E = 320000
D = 128
STRIDE = 4
M = N // STRIDE
DOWN = 128

NC = 2
NS = 16
NW = NC * NS
EPW = E // NW
K = 80
NB = EPW // K
NBUF = 3

MPAD = 2560
SM = MPAD // NS
NROWS = N + M
NPAD = 12544


def _matmul_body(x_ref, w_ref, o_ref):
    o_ref[...] = jnp.dot(x_ref[...], w_ref[...],
                         preferred_element_type=jnp.float32)


def _combine_body(p_ref, o_ref):
    o_ref[...] = p_ref[0] + p_ref[1]


def _sc_body(qall, idx4, zrows, out, idx_v, ep2_v, en2_v, ensc_v,
             q_v, b2_v, msg_v, acc,
             sem_i0, sem_i1, sem_i2, sem_q0, sem_q1, sem_q2,
             sem_b0, sem_b1, sem_b2, sem_s0, sem_s1, sem_s2):
    cid = lax.axis_index("c")
    sid = lax.axis_index("s")
    wid = sid * NC + cid
    sem_i = (sem_i0, sem_i1, sem_i2)
    sem_q = (sem_q0, sem_q1, sem_q2)
    sem_b = (sem_b0, sem_b1, sem_b2)
    sem_s = (sem_s0, sem_s1, sem_s2)

    pltpu.sync_copy(zrows, acc.at[pl.ds(sid * SM, SM)])
    plsc.subcore_barrier()

    def issue_idx(j, b):
        pltpu.async_copy(idx4.at[wid, j], idx_v.at[b], sem_i[b])

    def wait_idx(b):
        pltpu.make_async_copy(idx4.at[0, 0], idx_v.at[b], sem_i[b]).wait()

    def build_and_gather(b):
        for c in range(K // 16):
            s = pl.ds(c * 16, 16)
            ep2_v[b, s] = idx_v[b, 0, s]
            en = idx_v[b, 1, s]
            ensc_v[b, s] = en
            en2_v[b, s] = en + N
        pltpu.async_copy(qall.at[ep2_v.at[b]], q_v.at[b], sem_q[b])
        pltpu.async_copy(qall.at[en2_v.at[b]], b2_v.at[b], sem_b[b])

    def wait_gathers(b):
        pltpu.make_async_copy(qall.at[pl.ds(0, K)], q_v.at[b],
                              sem_q[b]).wait()
        pltpu.make_async_copy(qall.at[pl.ds(0, K)], b2_v.at[b],
                              sem_b[b]).wait()

    def wait_scatter(b):
        pltpu.make_async_copy(msg_v.at[b], acc.at[pl.ds(0, K)],
                              sem_s[b]).wait()

    def compute(b):
        @functools.partial(plsc.parallel_loop, 0, K, unroll=2)
        def _(r):
            for c in range(DOWN // 16):
                s = pl.ds(c * 16, 16)
                msg_v[b, r, s] = jnp.maximum(
                    q_v[b, r, s] - b2_v[b, r, s], 0.0)

    def scatter(b):
        pltpu.async_copy(msg_v.at[b], acc.at[ensc_v.at[b]],
                         sem_s[b], add=True)

    def do_block(j, b, scatter_wait, pf1, pf2):
        wait_gathers(b)
        if scatter_wait:
            wait_scatter(b)
        compute(b)
        scatter(b)
        if pf1 is not None:
            wait_idx(b)
            build_and_gather(b)
        if pf2 is not None:
            issue_idx(pf2, b)

    for b in range(NBUF):
        issue_idx(b, b)
    for b in range(NBUF):
        wait_idx(b)
        build_and_gather(b)
        issue_idx(b + NBUF, b)
    for b in range(NBUF):
        do_block(b, b, False, b + NBUF, b + 2 * NBUF)

    def group(i, carry):
        for b in range(NBUF):
            j = i * NBUF + b
            do_block(j, b, True, j + NBUF, j + 2 * NBUF)
        return carry

    lax.fori_loop(1, (NB - 8) // NBUF, group, 0)

    for j in range(NB - 8, NB):
        b = j % NBUF
        do_block(j, b, True,
                 j + NBUF if j + NBUF < NB else None,
                 j + 2 * NBUF if j + 2 * NBUF < NB else None)
    for j in range(NB - NBUF, NB):
        wait_scatter(j % NBUF)

    plsc.subcore_barrier()
    pltpu.sync_copy(acc.at[pl.ds(sid * SM, SM)],
                    out.at[cid, pl.ds(sid * SM, SM)])


_sc_kernel = functools.partial(
    pl.kernel,
    out_type=jax.ShapeDtypeStruct((NC, MPAD, DOWN), jnp.float32),
    mesh=plsc.VectorSubcoreMesh(core_axis_name="c", subcore_axis_name="s"),
    scratch_types=[
        pltpu.VMEM((NBUF, 2, K), jnp.int32),
        pltpu.VMEM((NBUF, K), jnp.int32),
        pltpu.VMEM((NBUF, K), jnp.int32),
        pltpu.VMEM((NBUF, K), jnp.int32),
        pltpu.VMEM((NBUF, K, DOWN), jnp.float32),
        pltpu.VMEM((NBUF, K, DOWN), jnp.float32),
        pltpu.VMEM((NBUF, K, DOWN), jnp.float32),
        pltpu.VMEM_SHARED((MPAD, DOWN), jnp.float32),
    ] + [pltpu.SemaphoreType.DMA] * 12,
)(_sc_body)


@jax.jit
def kernel(point_bxyz, point_feat, e_point, e_new, W, b):
    new_bxyz = point_bxyz[::STRIDE]
    xyz = point_bxyz[:, 1:4]
    nxyz = new_bxyz[:, 1:4]

    x_top = jnp.concatenate(
        [point_feat, xyz, jnp.ones((N, 1), jnp.float32)], axis=1)
    x_new = jnp.concatenate(
        [jnp.zeros((M, D), jnp.float32), nxyz,
         jnp.zeros((M, 1), jnp.float32)], axis=1)
    x_pad = jnp.zeros((NPAD - NROWS, D + 4), jnp.float32)
    x_all = jnp.concatenate([x_top, x_new, x_pad], axis=0)
    w_cat = jnp.concatenate([W, b[None, :]], axis=0)

    qall = pl.pallas_call(
        _matmul_body,
        out_shape=jax.ShapeDtypeStruct((NPAD, DOWN), jnp.float32),
    )(x_all, w_cat)

    idx4 = jnp.stack(
        [e_point.reshape(NW, NB, K), e_new.reshape(NW, NB, K)], axis=2)
    zrows = jnp.zeros((SM, DOWN), jnp.float32)

    partials = _sc_kernel(qall, idx4, zrows)

    feat_pad = pl.pallas_call(
        _combine_body,
        out_shape=jax.ShapeDtypeStruct((MPAD, DOWN), jnp.float32),
    )(partials)
    new_feat = feat_pad[:M]

    return (new_bxyz, new_feat, e_point, e_new)


# speedup vs baseline: 127.0301x; 1.1854x over previous
import functools

import jax
import jax.numpy as jnp
from jax import lax
from jax.experimental import pallas as pl
from jax.experimental.pallas import tpu as pltpu
from jax.experimental.pallas import tpu_sc as plsc

N = 10000
E = 320000
D = 128
STRIDE = 4
M = N // STRIDE
DOWN = 128

NC = 2
NS = 16
NW = NC * NS
EPW = E // NW
K = 80
NB = EPW // K
NBUF = 3

MPAD = 2560
SM = MPAD // NS
NROWS = N + M
NPAD = 12544


def _matmul_body(feat_ref, bxyz_ref, nbxyz_ref, w1_ref, w2_ref, b_ref,
                 o_ref):
    o_ref[0:N, :] = (
        jnp.dot(feat_ref[...], w1_ref[...],
                preferred_element_type=jnp.float32)
        + jnp.dot(bxyz_ref[...], w2_ref[...],
                  preferred_element_type=jnp.float32)
        + b_ref[...])
    o_ref[N:NPAD, :] = jnp.dot(nbxyz_ref[...], w2_ref[...],
                               preferred_element_type=jnp.float32)


def _combine_body(p_ref, o_ref):
    o_ref[...] = p_ref[0] + p_ref[1]


def _sc_body(qall, idx4, zrows, out, idx_v, ep2_v, en2_v, ensc_v,
             q_v, b2_v, msg_v, acc,
             sem_i0, sem_i1, sem_i2, sem_q0, sem_q1, sem_q2,
             sem_b0, sem_b1, sem_b2, sem_s0, sem_s1, sem_s2):
    cid = lax.axis_index("c")
    sid = lax.axis_index("s")
    wid = sid * NC + cid
    sem_i = (sem_i0, sem_i1, sem_i2)
    sem_q = (sem_q0, sem_q1, sem_q2)
    sem_b = (sem_b0, sem_b1, sem_b2)
    sem_s = (sem_s0, sem_s1, sem_s2)

    pltpu.sync_copy(zrows, acc.at[pl.ds(sid * SM, SM)])
    plsc.subcore_barrier()

    def issue_idx(j, b):
        pltpu.async_copy(idx4.at[wid, j], idx_v.at[b], sem_i[b])

    def wait_idx(b):
        pltpu.make_async_copy(idx4.at[0, 0], idx_v.at[b], sem_i[b]).wait()

    def build_and_gather(b, s6):
        for c in range(K // 16):
            s = pl.ds(c * 16, 16)
            ep2_v[b, s] = idx_v[b, 0, s]
            en = idx_v[b, 1, s]
            ensc_v[s6, s] = en
            en2_v[b, s] = en + N
        pltpu.async_copy(qall.at[ep2_v.at[b]], q_v.at[b], sem_q[b])
        pltpu.async_copy(qall.at[en2_v.at[b]], b2_v.at[b], sem_b[b])

    def wait_gathers(b):
        pltpu.make_async_copy(qall.at[pl.ds(0, K)], q_v.at[b],
                              sem_q[b]).wait()
        pltpu.make_async_copy(qall.at[pl.ds(0, K)], b2_v.at[b],
                              sem_b[b]).wait()

    def wait_scatter(b):
        pltpu.make_async_copy(msg_v.at[b], acc.at[pl.ds(0, K)],
                              sem_s[b]).wait()

    def compute(b):
        @functools.partial(plsc.parallel_loop, 0, K, unroll=2)
        def _(r):
            for c in range(DOWN // 16):
                s = pl.ds(c * 16, 16)
                msg_v[b, r, s] = jnp.maximum(
                    q_v[b, r, s] - b2_v[b, r, s], 0.0)

    def scatter(b, b6):
        pltpu.sync_copy(msg_v.at[b], acc.at[ensc_v.at[b6]], add=True)

    def do_block(j, b, b6, scatter_wait, pf1, pf2):
        wait_gathers(b)
        compute(b)
        scatter(b, b6)
        if pf1 is not None:
            wait_idx(b)
            build_and_gather(b, (b6 + NBUF) % (2 * NBUF))
        if pf2 is not None:
            issue_idx(pf2, b)

    for b in range(NBUF):
        issue_idx(b, b)
    for b in range(NBUF):
        wait_idx(b)
        build_and_gather(b, b)
        issue_idx(b + NBUF, b)
    for j in range(2 * NBUF):
        do_block(j, j % NBUF, j, j >= NBUF, j + NBUF, j + 2 * NBUF)

    def group(i, carry):
        for r in range(2 * NBUF):
            j = i * 2 * NBUF + r
            do_block(j, r % NBUF, r, True, j + NBUF, j + 2 * NBUF)
        return carry

    lax.fori_loop(1, (NB - 2 * NBUF) // (2 * NBUF), group, 0)

    for j in range(((NB - 2 * NBUF) // (2 * NBUF)) * 2 * NBUF, NB):
        do_block(j, j % NBUF, j % (2 * NBUF), True,
                 j + NBUF if j + NBUF < NB else None,
                 j + 2 * NBUF if j + 2 * NBUF < NB else None)

    plsc.subcore_barrier()
    pltpu.sync_copy(acc.at[pl.ds(sid * SM, SM)],
                    out.at[cid, pl.ds(sid * SM, SM)])


_sc_kernel = functools.partial(
    pl.kernel,
    out_type=jax.ShapeDtypeStruct((NC, MPAD, DOWN), jnp.float32),
    mesh=plsc.VectorSubcoreMesh(core_axis_name="c", subcore_axis_name="s",
                                num_cores=NC, num_subcores=NS),
    scratch_types=[
        pltpu.VMEM((NBUF, 2, K), jnp.int32),
        pltpu.VMEM((NBUF, K), jnp.int32),
        pltpu.VMEM((NBUF, K), jnp.int32),
        pltpu.VMEM((2 * NBUF, K), jnp.int32),
        pltpu.VMEM((NBUF, K, DOWN), jnp.float32),
        pltpu.VMEM((NBUF, K, DOWN), jnp.float32),
        pltpu.VMEM((NBUF, K, DOWN), jnp.float32),
        pltpu.VMEM_SHARED((MPAD, DOWN), jnp.float32),
    ] + [pltpu.SemaphoreType.DMA] * 12,
)(_sc_body)


@jax.jit
def kernel(point_bxyz, point_feat, e_point, e_new, W, b):
    new_bxyz = point_bxyz[::STRIDE]
    nbxyz_pad = jnp.concatenate(
        [new_bxyz, jnp.zeros((NPAD - N - M, 4), jnp.float32)], axis=0)
    w1 = W[:D]
    w2 = jnp.concatenate([jnp.zeros((1, DOWN), jnp.float32), W[D:]], axis=0)

    qall = pl.pallas_call(
        _matmul_body,
        out_shape=jax.ShapeDtypeStruct((NPAD, DOWN), jnp.float32),
    )(point_feat, point_bxyz, nbxyz_pad, w1, w2, b[None, :])

    idx4 = jnp.stack(
        [e_point.reshape(NW, NB, K), e_new.reshape(NW, NB, K)], axis=2)
    zrows = jnp.zeros((SM, DOWN), jnp.float32)

    partials = _sc_kernel(qall, idx4, zrows)

    feat_pad = pl.pallas_call(
        _combine_body,
        out_shape=jax.ShapeDtypeStruct((MPAD, DOWN), jnp.float32),
    )(partials)
    new_feat = feat_pad[:M]

    return (new_bxyz, new_feat, e_point, e_new)
